# baseline (speedup 1.0000x reference)
import functools

import jax
import jax.numpy as jnp
from jax import lax
from jax.experimental import pallas as pl
from jax.experimental.pallas import tpu as pltpu
from jax.experimental.pallas import tpu_sc as plsc

BATCH = 16384
EMBED_DIM = 64
NUM_FEATURES = 128
NUM_GENES = 100000
NUM_DISEASES = 1000
HID = 128

NUM_CORES = 2
NUM_SUBCORES = 16
NUM_WORKERS = NUM_CORES * NUM_SUBCORES
B_PER_W = BATCH // NUM_WORKERS
IDX_CHUNK = 128
N_CHUNKS = B_PER_W // IDX_CHUNK

GBLK = 8192
N_GBLK = -(-NUM_GENES // GBLK)


def _pg_body(tT_ref, w_ref, out_ref):
    out_ref[...] = lax.dot_general(
        tT_ref[...], w_ref[...],
        dimension_numbers=(((0,), (0,)), ((), ())),
        preferred_element_type=jnp.float32)


def _pd_body(tT_ref, w_ref, b_ref, out_ref):
    out_ref[...] = lax.dot_general(
        tT_ref[...], w_ref[...],
        dimension_numbers=(((0,), (0,)), ((), ())),
        preferred_element_type=jnp.float32) + b_ref[...]


def _build_products(gene_table, disease_table, W1, b1):
    gT = jnp.transpose(gene_table)
    dT = jnp.transpose(disease_table)
    w1g = W1[:EMBED_DIM]
    w1d = W1[EMBED_DIM:2 * EMBED_DIM]
    pg = pl.pallas_call(
        _pg_body,
        grid=(N_GBLK,),
        in_specs=[
            pl.BlockSpec((EMBED_DIM, GBLK), lambda i: (0, i)),
            pl.BlockSpec((EMBED_DIM, HID), lambda i: (0, 0)),
        ],
        out_specs=pl.BlockSpec((GBLK, HID), lambda i: (i, 0)),
        out_shape=jax.ShapeDtypeStruct((NUM_GENES, HID), jnp.float32),
    )(gT, w1g)
    pd = pl.pallas_call(
        _pd_body,
        out_shape=jax.ShapeDtypeStruct((NUM_DISEASES, HID), jnp.float32),
    )(dT, w1d, b1)
    return pg, pd


def _gather_body(pg_tab, pd_tab, gid_hbm, did_hbm, pre_out, idx_g, idx_d,
                 rows, sem):
    wid = lax.axis_index("s") * NUM_CORES + lax.axis_index("c")
    base = wid * B_PER_W
    pltpu.sync_copy(gid_hbm.at[wid], idx_g)
    pltpu.sync_copy(did_hbm.at[wid], idx_d)
    copies = []
    for j in range(N_CHUNKS):
        copies.append(pltpu.async_copy(
            pg_tab.at[idx_g.at[j]],
            rows.at[pl.ds(j * IDX_CHUNK, IDX_CHUNK)], sem))
    for c in copies:
        c.wait()
    copies = []
    for j in range(N_CHUNKS):
        copies.append(pltpu.async_copy(
            pd_tab.at[idx_d.at[j]],
            rows.at[pl.ds(j * IDX_CHUNK, IDX_CHUNK)], sem, add=True))
    for c in copies:
        c.wait()
    pltpu.sync_copy(rows, pre_out.at[pl.ds(base, B_PER_W)])


def _sc_gather(pg, pd, gid, did):
    mesh = plsc.VectorSubcoreMesh(core_axis_name="c", subcore_axis_name="s")
    out_type = jax.ShapeDtypeStruct((BATCH, HID), jnp.float32)
    scratch = [
        pltpu.VMEM((N_CHUNKS, IDX_CHUNK), jnp.int32),
        pltpu.VMEM((N_CHUNKS, IDX_CHUNK), jnp.int32),
        pltpu.VMEM((B_PER_W, HID), jnp.float32),
        pltpu.SemaphoreType.DMA,
    ]
    run = pl.kernel(_gather_body, out_type=out_type, mesh=mesh,
                    scratch_types=scratch)
    return run(pg, pd, gid, did)


BLK = 2048
N_BLK = BATCH // BLK


def _hx_body(x_ref, w1x_ref, hx_ref):
    hx_ref[...] = jnp.dot(x_ref[...], w1x_ref[...],
                          preferred_element_type=jnp.float32)


def _mlp_body(pre_ref, hx_ref, gamma_ref, beta_ref, w2_ref, b2_ref,
              w3_ref, b3_ref, out_ref, h_ref, stats_ref):
    s = pl.program_id(0)
    i = s % N_BLK

    @pl.when(s < N_BLK)
    def _phase1():
        h = pre_ref[...] + hx_ref[...]
        h_ref[pl.ds(i * BLK, BLK), :] = h
        part = jnp.concatenate(
            [jnp.sum(h, axis=0, keepdims=True),
             jnp.sum(h * h, axis=0, keepdims=True)], axis=0)

        @pl.when(s == 0)
        def _():
            stats_ref[...] = part

        @pl.when(s != 0)
        def _():
            stats_ref[...] += part

    @pl.when(s >= N_BLK)
    def _phase2():
        inv_n = 1.0 / BATCH
        mean = stats_ref[0:1, :] * inv_n
        var = stats_ref[1:2, :] * inv_n - mean * mean
        scale = lax.rsqrt(var + 1e-5) * gamma_ref[...]
        shift = beta_ref[...] - mean * scale
        h = jnp.maximum(h_ref[pl.ds(i * BLK, BLK), :] * scale + shift, 0.0)
        h2 = jnp.maximum(
            jnp.dot(h, w2_ref[...], preferred_element_type=jnp.float32)
            + b2_ref[...], 0.0)
        z = (jnp.dot(h2, w3_ref[...], preferred_element_type=jnp.float32)
             + b3_ref[...])
        out_ref[...] = jax.nn.sigmoid(z).reshape((BLK,))


def _tc_mlp(pre, hx, gamma, beta, w2, b2, w3, b3):
    row_blk = lambda s: (jnp.where(s < N_BLK, s, 0), 0)
    fixed = lambda s: (0, 0)
    return pl.pallas_call(
        _mlp_body,
        grid=(2 * N_BLK,),
        in_specs=[
            pl.BlockSpec((BLK, HID), row_blk),
            pl.BlockSpec((BLK, HID), row_blk),
            pl.BlockSpec((1, HID), fixed),
            pl.BlockSpec((1, HID), fixed),
            pl.BlockSpec((HID, 64), fixed),
            pl.BlockSpec((1, 64), fixed),
            pl.BlockSpec((64, 1), fixed),
            pl.BlockSpec((1, 1), fixed),
        ],
        out_specs=pl.BlockSpec((BLK,), lambda s: (s % N_BLK,)),
        out_shape=jax.ShapeDtypeStruct((BATCH,), jnp.float32),
        scratch_shapes=[
            pltpu.VMEM((BATCH, HID), jnp.float32),
            pltpu.VMEM((2, HID), jnp.float32),
        ],
    )(pre, hx, gamma, beta, w2, b2, w3, b3).reshape(BATCH, 1)


def kernel(gene_id, disease_id, explicit_features, gene_table, disease_table,
           W1, b1, gamma, beta, W2, b2, W3, b3):
    gid = gene_id.astype(jnp.int32).reshape(NUM_WORKERS, N_CHUNKS, IDX_CHUNK)
    did = disease_id.astype(jnp.int32).reshape(NUM_WORKERS, N_CHUNKS, IDX_CHUNK)
    pg, pd = _build_products(gene_table, disease_table, W1,
                             b1.reshape(1, -1))
    pre = _sc_gather(pg, pd, gid, did)
    w1x = W1[2 * EMBED_DIM:]
    hx = pl.pallas_call(
        _hx_body,
        grid=(N_BLK,),
        in_specs=[
            pl.BlockSpec((BLK, NUM_FEATURES), lambda i: (i, 0)),
            pl.BlockSpec((NUM_FEATURES, HID), lambda i: (0, 0)),
        ],
        out_specs=pl.BlockSpec((BLK, HID), lambda i: (i, 0)),
        out_shape=jax.ShapeDtypeStruct((BATCH, HID), jnp.float32),
    )(explicit_features, w1x)
    return _tc_mlp(pre, hx,
                   gamma.reshape(1, -1), beta.reshape(1, -1),
                   W2, b2.reshape(1, -1), W3, b3.reshape(1, -1))


# speedup vs baseline: 1.0643x; 1.0643x over previous
import functools

import jax
import jax.numpy as jnp
from jax import lax
from jax.experimental import pallas as pl
from jax.experimental.pallas import tpu as pltpu
from jax.experimental.pallas import tpu_sc as plsc

BATCH = 16384
EMBED_DIM = 64
NUM_FEATURES = 128
NUM_GENES = 100000
NUM_DISEASES = 1000
HID = 128

NUM_CORES = 2
NUM_SUBCORES = 16
NUM_WORKERS = NUM_CORES * NUM_SUBCORES
B_PER_W = BATCH // NUM_WORKERS
IDX_CHUNK = 128
N_CHUNKS = B_PER_W // IDX_CHUNK

GBLK = 8192
N_GBLK = -(-NUM_GENES // GBLK)


def _pg_body(tT_ref, w_ref, out_ref):
    out_ref[...] = lax.dot_general(
        tT_ref[...], w_ref[...],
        dimension_numbers=(((0,), (0,)), ((), ())),
        preferred_element_type=jnp.float32)


def _pd_body(tT_ref, w_ref, b_ref, out_ref):
    out_ref[...] = lax.dot_general(
        tT_ref[...], w_ref[...],
        dimension_numbers=(((0,), (0,)), ((), ())),
        preferred_element_type=jnp.float32) + b_ref[...]


def _build_products(gene_table, disease_table, W1, b1):
    gT = jnp.transpose(gene_table)
    dT = jnp.transpose(disease_table)
    w1g = W1[:EMBED_DIM]
    w1d = W1[EMBED_DIM:2 * EMBED_DIM]
    pg = pl.pallas_call(
        _pg_body,
        grid=(N_GBLK,),
        in_specs=[
            pl.BlockSpec((EMBED_DIM, GBLK), lambda i: (0, i)),
            pl.BlockSpec((EMBED_DIM, HID), lambda i: (0, 0)),
        ],
        out_specs=pl.BlockSpec((GBLK, HID), lambda i: (i, 0)),
        out_shape=jax.ShapeDtypeStruct((NUM_GENES, HID), jnp.float32),
    )(gT, w1g)
    pd = pl.pallas_call(
        _pd_body,
        out_shape=jax.ShapeDtypeStruct((NUM_DISEASES, HID), jnp.float32),
    )(dT, w1d, b1)
    return pg, pd


def _gather_body(pg_tab, pd_tab, gid_hbm, did_hbm, pre_out, idx_g, idx_d,
                 rows, semg, semd, semw):
    wid = lax.axis_index("s") * NUM_CORES + lax.axis_index("c")
    base = wid * B_PER_W
    pltpu.sync_copy(gid_hbm.at[wid], idx_g)
    pltpu.sync_copy(did_hbm.at[wid], idx_d)
    gcopies = [pltpu.async_copy(
        pg_tab.at[idx_g.at[j]],
        rows.at[pl.ds(j * IDX_CHUNK, IDX_CHUNK)], semg)
        for j in range(N_CHUNKS)]
    dcopies = []
    wcopies = []
    for j in range(N_CHUNKS):
        gcopies[j].wait()
        dcopies.append(pltpu.async_copy(
            pd_tab.at[idx_d.at[j]],
            rows.at[pl.ds(j * IDX_CHUNK, IDX_CHUNK)], semd, add=True))
        if j > 0:
            dcopies[j - 1].wait()
            wcopies.append(pltpu.async_copy(
                rows.at[pl.ds((j - 1) * IDX_CHUNK, IDX_CHUNK)],
                pre_out.at[pl.ds(base + (j - 1) * IDX_CHUNK, IDX_CHUNK)],
                semw))
    dcopies[N_CHUNKS - 1].wait()
    wcopies.append(pltpu.async_copy(
        rows.at[pl.ds((N_CHUNKS - 1) * IDX_CHUNK, IDX_CHUNK)],
        pre_out.at[pl.ds(base + (N_CHUNKS - 1) * IDX_CHUNK, IDX_CHUNK)],
        semw))
    for c in wcopies:
        c.wait()


def _sc_gather(pg, pd, gid, did):
    mesh = plsc.VectorSubcoreMesh(core_axis_name="c", subcore_axis_name="s")
    out_type = jax.ShapeDtypeStruct((BATCH, HID), jnp.float32)
    scratch = [
        pltpu.VMEM((N_CHUNKS, IDX_CHUNK), jnp.int32),
        pltpu.VMEM((N_CHUNKS, IDX_CHUNK), jnp.int32),
        pltpu.VMEM((B_PER_W, HID), jnp.float32),
        pltpu.SemaphoreType.DMA,
        pltpu.SemaphoreType.DMA,
        pltpu.SemaphoreType.DMA,
    ]
    run = pl.kernel(_gather_body, out_type=out_type, mesh=mesh,
                    scratch_types=scratch)
    return run(pg, pd, gid, did)


BLK = 2048
N_BLK = BATCH // BLK


def _mlp_body(pre_ref, x_ref, w1x_ref, gamma_ref, beta_ref, w2_ref, b2_ref,
              w3_ref, b3_ref, out_ref, h_ref, stats_ref):
    s = pl.program_id(0)
    i = s % N_BLK

    @pl.when(s < N_BLK)
    def _phase1():
        h = pre_ref[...] + jnp.dot(x_ref[...], w1x_ref[...],
                                   preferred_element_type=jnp.float32)
        h_ref[pl.ds(i * BLK, BLK), :] = h
        part = jnp.concatenate(
            [jnp.sum(h, axis=0, keepdims=True),
             jnp.sum(h * h, axis=0, keepdims=True)], axis=0)

        @pl.when(s == 0)
        def _():
            stats_ref[...] = part

        @pl.when(s != 0)
        def _():
            stats_ref[...] += part

    @pl.when(s >= N_BLK)
    def _phase2():
        inv_n = 1.0 / BATCH
        mean = stats_ref[0:1, :] * inv_n
        var = stats_ref[1:2, :] * inv_n - mean * mean
        scale = lax.rsqrt(var + 1e-5) * gamma_ref[...]
        shift = beta_ref[...] - mean * scale
        h = jnp.maximum(h_ref[pl.ds(i * BLK, BLK), :] * scale + shift, 0.0)
        h2 = jnp.maximum(
            jnp.dot(h, w2_ref[...], preferred_element_type=jnp.float32)
            + b2_ref[...], 0.0)
        z = (jnp.dot(h2, w3_ref[...], preferred_element_type=jnp.float32)
             + b3_ref[...])
        out_ref[...] = jax.nn.sigmoid(z).reshape((BLK,))


def _tc_mlp(pre, x, w1x, gamma, beta, w2, b2, w3, b3):
    row_blk = lambda s: (jnp.where(s < N_BLK, s, 0), 0)
    fixed = lambda s: (0, 0)
    return pl.pallas_call(
        _mlp_body,
        grid=(2 * N_BLK,),
        in_specs=[
            pl.BlockSpec((BLK, HID), row_blk),
            pl.BlockSpec((BLK, NUM_FEATURES), row_blk),
            pl.BlockSpec((NUM_FEATURES, HID), fixed),
            pl.BlockSpec((1, HID), fixed),
            pl.BlockSpec((1, HID), fixed),
            pl.BlockSpec((HID, 64), fixed),
            pl.BlockSpec((1, 64), fixed),
            pl.BlockSpec((64, 1), fixed),
            pl.BlockSpec((1, 1), fixed),
        ],
        out_specs=pl.BlockSpec((BLK,), lambda s: (s % N_BLK,)),
        out_shape=jax.ShapeDtypeStruct((BATCH,), jnp.float32),
        scratch_shapes=[
            pltpu.VMEM((BATCH, HID), jnp.float32),
            pltpu.VMEM((2, HID), jnp.float32),
        ],
    )(pre, x, w1x, gamma, beta, w2, b2, w3, b3).reshape(BATCH, 1)


def kernel(gene_id, disease_id, explicit_features, gene_table, disease_table,
           W1, b1, gamma, beta, W2, b2, W3, b3):
    gid = gene_id.astype(jnp.int32).reshape(NUM_WORKERS, N_CHUNKS, IDX_CHUNK)
    did = disease_id.astype(jnp.int32).reshape(NUM_WORKERS, N_CHUNKS, IDX_CHUNK)
    pg, pd = _build_products(gene_table, disease_table, W1,
                             b1.reshape(1, -1))
    pre = _sc_gather(pg, pd, gid, did)
    w1x = W1[2 * EMBED_DIM:]
    return _tc_mlp(pre, explicit_features, w1x,
                   gamma.reshape(1, -1), beta.reshape(1, -1),
                   W2, b2.reshape(1, -1), W3, b3.reshape(1, -1))
